# baseline (speedup 1.0000x reference)
import functools

import jax
import jax.numpy as jnp
from jax import lax
from jax.experimental import pallas as pl
from jax.experimental.pallas import tpu as pltpu
from jax.experimental.pallas import tpu_sc as plsc

N = 10000
NP = 10240
E = 320000
FH = 128
IDXW = 50
NTEC = 16
NPT = NP // NTEC
ER = E // IDXW
CPT = ER // NTEC
IB = 16
NSLOT = 6
LAG = 3

_mesh = plsc.VectorSubcoreMesh(core_axis_name="c", subcore_axis_name="s",
                               num_cores=2, num_subcores=NTEC)
_linear = pltpu.CompilerParams(use_tc_tiling_on_sc=False)


_DCPT = ER // 32
_DIB = 25
_DLAG = 8


def _deg_body(ei_hbm, ones_hbm, zeros_hbm, out_hbm, didx_v, ones_v, acc_sh,
              ssem, isem):
    c = lax.axis_index("c")
    s = lax.axis_index("s")
    dst_hbm = ei_hbm.at[1]
    pltpu.sync_copy(ones_hbm, ones_v)
    pltpu.sync_copy(zeros_hbm, acc_sh.at[pl.ds(s * NPT, NPT)])
    plsc.subcore_barrier()
    w = s * 2 + c
    base = w * _DCPT

    def fire_idx(i, sb):
        pltpu.make_async_copy(dst_hbm.at[pl.ds(base + i, _DIB)],
                              didx_v.at[sb], isem).start()

    def wait_idx():
        pltpu.make_async_copy(dst_hbm.at[pl.ds(0, _DIB)], didx_v.at[0],
                              isem).wait()

    def fire_s(i):
        sb = lax.rem(i // _DIB, 2)
        pltpu.make_async_copy(ones_v, acc_sh.at[didx_v.at[sb, lax.rem(i, _DIB)]],
                              ssem).start(add=True)

    def drain():
        pltpu.make_async_copy(zeros_hbm.at[pl.ds(0, IDXW)], ones_v, ssem).wait()

    fire_idx(0, 0)
    wait_idx()

    def step(i, carry):
        pl.when(jnp.logical_and(lax.rem(i, _DIB) == 0, i > 0))(wait_idx)
        pl.when(jnp.logical_and(lax.rem(i, _DIB) == _DLAG + 1,
                                i + 2 * _DIB - _DLAG - 1 <= _DCPT))(
            lambda: fire_idx(i + _DIB - _DLAG - 1, lax.rem(i // _DIB + 1, 2)))
        pl.when(i >= _DLAG)(drain)
        fire_s(i)
        return carry

    lax.fori_loop(0, _DCPT, step, 0)
    for _ in range(_DLAG):
        drain()
    plsc.subcore_barrier()
    pltpu.sync_copy(acc_sh.at[pl.ds(s * NPT, NPT)],
                    out_hbm.at[c, pl.ds(s * NPT, NPT)])


_deg_call = functools.partial(
    pl.kernel,
    out_type=jax.ShapeDtypeStruct((2, NP, 16), jnp.float32),
    mesh=_mesh,
    compiler_params=_linear,
    scratch_types=[
        pltpu.VMEM((2, _DIB, IDXW), jnp.int32),
        pltpu.VMEM((IDXW, 16), jnp.float32),
        pltpu.VMEM_SHARED((NP, 16), jnp.float32),
        pltpu.SemaphoreType.DMA,
        pltpu.SemaphoreType.DMA,
    ],
)(_deg_body)


def _agg_body(tab_hbm, ei_hbm, zeros_hbm, out_hbm,
              sidx_v, didx_v, rows_v, acc_sh, gsem, ssem, isem):
    c = lax.axis_index("c")
    s = lax.axis_index("s")
    src_hbm = ei_hbm.at[0]
    dst_hbm = ei_hbm.at[1]
    pltpu.sync_copy(zeros_hbm, acc_sh.at[pl.ds(s * NPT, NPT)])
    plsc.subcore_barrier()
    tab = tab_hbm.at[c]
    base = s * CPT
    zrow = zeros_hbm.at[pl.ds(0, IDXW)]

    def fire_idx(i, sb):
        pltpu.make_async_copy(src_hbm.at[pl.ds(base + i, IB)],
                              sidx_v.at[sb], isem).start()
        pltpu.make_async_copy(dst_hbm.at[pl.ds(base + i, IB)],
                              didx_v.at[sb], isem).start()

    def wait_idx():
        pltpu.make_async_copy(src_hbm.at[pl.ds(0, IB)], sidx_v.at[0], isem).wait()
        pltpu.make_async_copy(dst_hbm.at[pl.ds(0, IB)], didx_v.at[0], isem).wait()

    def fire_g(i):
        sb = lax.rem(i // IB, 2)
        pltpu.make_async_copy(tab.at[sidx_v.at[sb, lax.rem(i, IB)]],
                              rows_v.at[lax.rem(i, NSLOT)], gsem).start()

    def fire_s(i):
        sb = lax.rem(i // IB, 2)
        pltpu.make_async_copy(rows_v.at[lax.rem(i, NSLOT)],
                              acc_sh.at[didx_v.at[sb, lax.rem(i, IB)]],
                              ssem).start(add=True)

    def drain(sem):
        pltpu.make_async_copy(zrow, rows_v.at[0], sem).wait()

    fire_idx(0, 0)
    wait_idx()
    for j in range(LAG):
        fire_g(j)

    def step(i, carry):
        pl.when(lax.rem(i, IB) == 0)(wait_idx)
        pl.when(jnp.logical_and(lax.rem(i, IB) == LAG + 2,
                                i + 2 * IB - LAG - 2 <= CPT))(
            lambda: fire_idx(i + IB - LAG - 2, lax.rem(i // IB + 1, 2)))
        pl.when(i >= NSLOT)(lambda: drain(ssem))
        fire_g(i)
        drain(gsem)
        fire_s(i - LAG)
        return carry

    lax.fori_loop(LAG, CPT, step, 0)
    for j in range(LAG):
        drain(gsem)
        fire_s(CPT - LAG + j)
    for _ in range(NSLOT):
        drain(ssem)
    plsc.subcore_barrier()
    pltpu.sync_copy(acc_sh.at[pl.ds(s * NPT, NPT)],
                    out_hbm.at[c, pl.ds(s * NPT, NPT)])
    plsc.subcore_barrier()


_agg_call = functools.partial(
    pl.kernel,
    out_type=jax.ShapeDtypeStruct((2, NP, FH), jnp.float32),
    mesh=_mesh,
    compiler_params=_linear,
    scratch_types=[
        pltpu.VMEM((2, IB, IDXW), jnp.int32),
        pltpu.VMEM((2, IB, IDXW), jnp.int32),
        pltpu.VMEM((NSLOT, IDXW, FH), jnp.float32),
        pltpu.VMEM_SHARED((NP, FH), jnp.float32),
        pltpu.SemaphoreType.DMA,
        pltpu.SemaphoreType.DMA,
        pltpu.SemaphoreType.DMA,
    ],
)(_agg_body)


_R = 2048


def _dinv_of(degp):
    deg = jnp.sum(degp[0] + degp[1], axis=1, keepdims=True) + 1.0
    return lax.rsqrt(deg)


def _mm1_body(x_ref, w1_ref, degp_ref, u_ref):
    dinv = _dinv_of(degp_ref[...])
    h = jnp.dot(x_ref[...], w1_ref[...], preferred_element_type=jnp.float32)
    u = h * dinv
    u_ref[0] = u[:, :FH]
    u_ref[1] = u[:, FH:]


def _mm23_body(agg_ref, u_ref, degp_ref, w_ref, b_ref, o_ref, *, last):
    dinv = _dinv_of(degp_ref[...])
    t0 = (agg_ref[0] + u_ref[0]) * dinv
    t1 = (agg_ref[1] + u_ref[1]) * dinv
    h = jnp.concatenate([t0, t1], axis=1) + b_ref[...]
    h = jnp.maximum(h, 0.0)
    o = jnp.dot(h, w_ref[...], preferred_element_type=jnp.float32)
    if last:
        o_ref[...] = o
    else:
        o = o * dinv
        o_ref[0] = o[:, :FH]
        o_ref[1] = o[:, FH:]


def _mm_call(body, out_shape, out_spec, in_specs):
    return pl.pallas_call(
        body,
        grid=(NP // _R,),
        in_specs=in_specs,
        out_specs=out_spec,
        out_shape=out_shape,
    )


_spec_half = pl.BlockSpec((2, _R, FH), lambda i: (0, i, 0))
_spec_degp = pl.BlockSpec((2, _R, 16), lambda i: (0, i, 0))


def kernel(x, edge_index, W1, b1, W2, b2, Wfc, bfc):
    ei3 = edge_index.astype(jnp.int32).reshape(2, ER, IDXW)
    ones16 = jnp.full((IDXW, 16), 1.0 / 16, jnp.float32)
    zeros16 = jnp.zeros((NPT, 16), jnp.float32)
    zerosF = jnp.zeros((NPT, FH), jnp.float32)

    degp = _deg_call(ei3, ones16, zeros16)

    u1 = _mm_call(
        _mm1_body,
        jax.ShapeDtypeStruct((2, NP, FH), jnp.float32),
        _spec_half,
        [pl.BlockSpec((_R, 128), lambda i: (i, 0)),
         pl.BlockSpec((128, 256), lambda i: (0, 0)),
         _spec_degp],
    )(x, W1, degp)

    agg1 = _agg_call(u1, ei3, zerosF)

    u2 = _mm_call(
        functools.partial(_mm23_body, last=False),
        jax.ShapeDtypeStruct((2, NP, FH), jnp.float32),
        _spec_half,
        [_spec_half, _spec_half, _spec_degp,
         pl.BlockSpec((256, 256), lambda i: (0, 0)),
         pl.BlockSpec((1, 256), lambda i: (0, 0))],
    )(agg1, u1, degp, W2, b1.reshape(1, 256))

    agg2 = _agg_call(u2, ei3, zerosF)

    wfc_pad = jnp.zeros((256, 128), jnp.float32).at[:, :2].set(Wfc)
    outp = _mm_call(
        functools.partial(_mm23_body, last=True),
        jax.ShapeDtypeStruct((NP, 128), jnp.float32),
        pl.BlockSpec((_R, 128), lambda i: (i, 0)),
        [_spec_half, _spec_half, _spec_degp,
         pl.BlockSpec((256, 128), lambda i: (0, 0)),
         pl.BlockSpec((1, 256), lambda i: (0, 0))],
    )(agg2, u2, degp, wfc_pad, b2.reshape(1, 256))

    return outp[:N, :2] + bfc


# speedup vs baseline: 1.0719x; 1.0719x over previous
import functools

import jax
import jax.numpy as jnp
from jax import lax
from jax.experimental import pallas as pl
from jax.experimental.pallas import tpu as pltpu
from jax.experimental.pallas import tpu_sc as plsc

N = 10000
NP = 10240
E = 320000
FH = 128
IDXW = 80
NTEC = 16
NPT = NP // NTEC
ER = E // IDXW
CPT = ER // NTEC
IB = 10
NSLOT = 4
LAG = 2

_mesh = plsc.VectorSubcoreMesh(core_axis_name="c", subcore_axis_name="s",
                               num_cores=2, num_subcores=NTEC)
_linear = pltpu.CompilerParams(use_tc_tiling_on_sc=False)


_DCPT = ER // 32
_DIB = 25
_DLAG = 8


def _deg_body(ei_hbm, ones_hbm, zeros_hbm, out_hbm, didx_v, ones_v, acc_sh,
              ssem, isem):
    c = lax.axis_index("c")
    s = lax.axis_index("s")
    dst_hbm = ei_hbm.at[1]
    pltpu.sync_copy(ones_hbm, ones_v)
    pltpu.sync_copy(zeros_hbm, acc_sh.at[pl.ds(s * NPT, NPT)])
    plsc.subcore_barrier()
    w = s * 2 + c
    base = w * _DCPT

    def fire_idx(i, sb):
        pltpu.make_async_copy(dst_hbm.at[pl.ds(base + i, _DIB)],
                              didx_v.at[sb], isem).start()

    def wait_idx():
        pltpu.make_async_copy(dst_hbm.at[pl.ds(0, _DIB)], didx_v.at[0],
                              isem).wait()

    def fire_s(i):
        sb = lax.rem(i // _DIB, 2)
        pltpu.make_async_copy(ones_v, acc_sh.at[didx_v.at[sb, lax.rem(i, _DIB)]],
                              ssem).start(add=True)

    def drain():
        pltpu.make_async_copy(zeros_hbm.at[pl.ds(0, IDXW)], ones_v, ssem).wait()

    fire_idx(0, 0)
    wait_idx()

    def step(i, carry):
        pl.when(jnp.logical_and(lax.rem(i, _DIB) == 0, i > 0))(wait_idx)
        pl.when(jnp.logical_and(lax.rem(i, _DIB) == _DLAG + 1,
                                i + 2 * _DIB - _DLAG - 1 <= _DCPT))(
            lambda: fire_idx(i + _DIB - _DLAG - 1, lax.rem(i // _DIB + 1, 2)))
        pl.when(i >= _DLAG)(drain)
        fire_s(i)
        return carry

    lax.fori_loop(0, _DCPT, step, 0)
    for _ in range(_DLAG):
        drain()
    plsc.subcore_barrier()
    pltpu.sync_copy(acc_sh.at[pl.ds(s * NPT, NPT)],
                    out_hbm.at[c, pl.ds(s * NPT, NPT)])


_deg_call = functools.partial(
    pl.kernel,
    out_type=jax.ShapeDtypeStruct((2, NP, 16), jnp.float32),
    mesh=_mesh,
    compiler_params=_linear,
    scratch_types=[
        pltpu.VMEM((2, _DIB, IDXW), jnp.int32),
        pltpu.VMEM((IDXW, 16), jnp.float32),
        pltpu.VMEM_SHARED((NP, 16), jnp.float32),
        pltpu.SemaphoreType.DMA,
        pltpu.SemaphoreType.DMA,
    ],
)(_deg_body)


def _agg_body(tab_hbm, ei_hbm, zeros_hbm, out_hbm,
              sidx_v, didx_v, rows_v, acc_sh, gsem, ssem, isem):
    c = lax.axis_index("c")
    s = lax.axis_index("s")
    src_hbm = ei_hbm.at[0]
    dst_hbm = ei_hbm.at[1]
    pltpu.sync_copy(zeros_hbm, acc_sh.at[pl.ds(s * NPT, NPT)])
    plsc.subcore_barrier()
    tab = tab_hbm.at[c]
    base = s * CPT
    zrow = zeros_hbm.at[pl.ds(0, IDXW)]

    def fire_idx(i, sb):
        pltpu.make_async_copy(src_hbm.at[pl.ds(base + i, IB)],
                              sidx_v.at[sb], isem).start()
        pltpu.make_async_copy(dst_hbm.at[pl.ds(base + i, IB)],
                              didx_v.at[sb], isem).start()

    def wait_idx():
        pltpu.make_async_copy(src_hbm.at[pl.ds(0, IB)], sidx_v.at[0], isem).wait()
        pltpu.make_async_copy(dst_hbm.at[pl.ds(0, IB)], didx_v.at[0], isem).wait()

    def fire_g(i):
        sb = lax.rem(i // IB, 2)
        pltpu.make_async_copy(tab.at[sidx_v.at[sb, lax.rem(i, IB)]],
                              rows_v.at[lax.rem(i, NSLOT)], gsem).start()

    def fire_s(i):
        sb = lax.rem(i // IB, 2)
        pltpu.make_async_copy(rows_v.at[lax.rem(i, NSLOT)],
                              acc_sh.at[didx_v.at[sb, lax.rem(i, IB)]],
                              ssem).start(add=True)

    def drain(sem):
        pltpu.make_async_copy(zrow, rows_v.at[0], sem).wait()

    fire_idx(0, 0)
    wait_idx()
    for j in range(LAG):
        fire_g(j)

    def step(i, carry):
        pl.when(lax.rem(i, IB) == 0)(wait_idx)
        pl.when(jnp.logical_and(lax.rem(i, IB) == LAG + 2,
                                i + 2 * IB - LAG - 2 <= CPT))(
            lambda: fire_idx(i + IB - LAG - 2, lax.rem(i // IB + 1, 2)))
        pl.when(i >= NSLOT)(lambda: drain(ssem))
        fire_g(i)
        drain(gsem)
        fire_s(i - LAG)
        return carry

    lax.fori_loop(LAG, CPT, step, 0)
    for j in range(LAG):
        drain(gsem)
        fire_s(CPT - LAG + j)
    for _ in range(NSLOT):
        drain(ssem)
    plsc.subcore_barrier()
    pltpu.sync_copy(acc_sh.at[pl.ds(s * NPT, NPT)],
                    out_hbm.at[c, pl.ds(s * NPT, NPT)])
    plsc.subcore_barrier()


_agg_call = functools.partial(
    pl.kernel,
    out_type=jax.ShapeDtypeStruct((2, NP, FH), jnp.float32),
    mesh=_mesh,
    compiler_params=_linear,
    scratch_types=[
        pltpu.VMEM((2, IB, IDXW), jnp.int32),
        pltpu.VMEM((2, IB, IDXW), jnp.int32),
        pltpu.VMEM((NSLOT, IDXW, FH), jnp.float32),
        pltpu.VMEM_SHARED((NP, FH), jnp.float32),
        pltpu.SemaphoreType.DMA,
        pltpu.SemaphoreType.DMA,
        pltpu.SemaphoreType.DMA,
    ],
)(_agg_body)


_R = 2048


def _dinv_of(degp):
    deg = jnp.sum(degp[0] + degp[1], axis=1, keepdims=True) + 1.0
    return lax.rsqrt(deg)


def _mm1_body(x_ref, w1_ref, degp_ref, u_ref):
    dinv = _dinv_of(degp_ref[...])
    h = jnp.dot(x_ref[...], w1_ref[...], preferred_element_type=jnp.float32)
    u = h * dinv
    u_ref[0] = u[:, :FH]
    u_ref[1] = u[:, FH:]


def _mm23_body(agg_ref, u_ref, degp_ref, w_ref, b_ref, o_ref, *, last):
    dinv = _dinv_of(degp_ref[...])
    t0 = (agg_ref[0] + u_ref[0]) * dinv
    t1 = (agg_ref[1] + u_ref[1]) * dinv
    h = jnp.concatenate([t0, t1], axis=1) + b_ref[...]
    h = jnp.maximum(h, 0.0)
    o = jnp.dot(h, w_ref[...], preferred_element_type=jnp.float32)
    if last:
        o_ref[...] = o
    else:
        o = o * dinv
        o_ref[0] = o[:, :FH]
        o_ref[1] = o[:, FH:]


def _mm_call(body, out_shape, out_spec, in_specs):
    return pl.pallas_call(
        body,
        grid=(NP // _R,),
        in_specs=in_specs,
        out_specs=out_spec,
        out_shape=out_shape,
    )


_spec_half = pl.BlockSpec((2, _R, FH), lambda i: (0, i, 0))
_spec_degp = pl.BlockSpec((2, _R, 16), lambda i: (0, i, 0))


def kernel(x, edge_index, W1, b1, W2, b2, Wfc, bfc):
    ei3 = edge_index.astype(jnp.int32).reshape(2, ER, IDXW)
    ones16 = jnp.full((IDXW, 16), 1.0 / 16, jnp.float32)
    zeros16 = jnp.zeros((NPT, 16), jnp.float32)
    zerosF = jnp.zeros((NPT, FH), jnp.float32)

    degp = _deg_call(ei3, ones16, zeros16)

    u1 = _mm_call(
        _mm1_body,
        jax.ShapeDtypeStruct((2, NP, FH), jnp.float32),
        _spec_half,
        [pl.BlockSpec((_R, 128), lambda i: (i, 0)),
         pl.BlockSpec((128, 256), lambda i: (0, 0)),
         _spec_degp],
    )(x, W1, degp)

    agg1 = _agg_call(u1, ei3, zerosF)

    u2 = _mm_call(
        functools.partial(_mm23_body, last=False),
        jax.ShapeDtypeStruct((2, NP, FH), jnp.float32),
        _spec_half,
        [_spec_half, _spec_half, _spec_degp,
         pl.BlockSpec((256, 256), lambda i: (0, 0)),
         pl.BlockSpec((1, 256), lambda i: (0, 0))],
    )(agg1, u1, degp, W2, b1.reshape(1, 256))

    agg2 = _agg_call(u2, ei3, zerosF)

    wfc_pad = jnp.zeros((256, 128), jnp.float32).at[:, :2].set(Wfc)
    outp = _mm_call(
        functools.partial(_mm23_body, last=True),
        jax.ShapeDtypeStruct((NP, 128), jnp.float32),
        pl.BlockSpec((_R, 128), lambda i: (i, 0)),
        [_spec_half, _spec_half, _spec_degp,
         pl.BlockSpec((256, 128), lambda i: (0, 0)),
         pl.BlockSpec((1, 256), lambda i: (0, 0))],
    )(agg2, u2, degp, wfc_pad, b2.reshape(1, 256))

    return outp[:N, :2] + bfc
